# baseline (speedup 1.0000x reference)
import jax
import jax.numpy as jnp
from jax import lax
from jax.experimental import pallas as pl
from jax.experimental.pallas import tpu as pltpu
from jax.experimental.pallas import tpu_sc as plsc

_N = 16
_M = 64
_B = 1048576

_NW = 32
_BPW = _B // _NW
_CH = 2048
_NCHUNK = _BPW // _CH
_NBUF = 3
_L = 16


def _logsm_body(mt_ref, out_ref):
    x = mt_ref[...]
    mx = jnp.max(x, axis=0, keepdims=True)
    s = x - mx
    lse = jnp.log(jnp.sum(jnp.exp(s), axis=0, keepdims=True))
    out_ref[...] = s - lse


def _make_table(matrix):
    return pl.pallas_call(
        _logsm_body,
        out_shape=jax.ShapeDtypeStruct((_M, _N), jnp.float32),
    )(matrix.T)


def _gather_body(table_hbm, xt_hbm, out_hbm, table_v, idx_v, rows_v, *sems):
    si = sems[0:_NBUF]
    sw = sems[_NBUF:2 * _NBUF]
    wid = lax.axis_index("s") * 2 + lax.axis_index("c")
    base = wid * _BPW

    pltpu.sync_copy(table_hbm, table_v)

    def start_idx(b, i):
        return pltpu.async_copy(
            xt_hbm.at[pl.ds(base + i * _CH, _CH)], idx_v.at[b], si[b])

    def start_write(b, i):
        return pltpu.async_copy(
            rows_v.at[b], out_hbm.at[pl.ds((base + i * _CH) * _N, _CH * _N)],
            sw[b])

    iota16 = lax.iota(jnp.int32, _L)

    def compute_chunk(b):
        rows = rows_v.at[b]

        def group(g, carry):
            o = g * _L
            tv = idx_v[b, pl.ds(o, _L)] * _N
            av = (o * _N) + iota16 * _N
            for n in range(_N):
                vals = plsc.load_gather(table_v, [tv + n])
                plsc.store_scatter(rows, [av + n], vals)
            return carry

        lax.fori_loop(0, _CH // _L, group, 0)

    h_idx = [None] * _NBUF
    h_w = [None] * _NBUF
    for i in range(_NBUF):
        h_idx[i] = start_idx(i, i)
    for i in range(_NCHUNK):
        b = i % _NBUF
        h_idx[b].wait()
        if i >= _NBUF:
            h_w[b].wait()
        compute_chunk(b)
        h_w[b] = start_write(b, i)
        if i + _NBUF < _NCHUNK:
            h_idx[b] = start_idx(b, i + _NBUF)
    for b in range(_NBUF):
        h_w[b].wait()


def kernel(matrix, x_t):
    table = _make_table(matrix).reshape(_M * _N)
    f = pl.kernel(
        _gather_body,
        out_type=jax.ShapeDtypeStruct((_B * _N,), jnp.float32),
        mesh=plsc.VectorSubcoreMesh(core_axis_name="c", subcore_axis_name="s"),
        scratch_types=[
            pltpu.VMEM((_M * _N,), jnp.float32),
            pltpu.VMEM((_NBUF, _CH), jnp.int32),
            pltpu.VMEM((_NBUF, _CH * _N), jnp.float32),
        ] + [pltpu.SemaphoreType.DMA] * (2 * _NBUF),
        compiler_params=pltpu.CompilerParams(
            use_tc_tiling_on_sc=False, needs_layout_passes=False),
    )
    return f(table, x_t).reshape(_B, _N)


# speedup vs baseline: 1.3295x; 1.3295x over previous
import jax
import jax.numpy as jnp
from jax import lax
from jax.experimental import pallas as pl
from jax.experimental.pallas import tpu as pltpu
from jax.experimental.pallas import tpu_sc as plsc

_N = 16
_M = 64
_B = 1048576

_NW = 32
_BPW = _B // _NW
_CH = 2048
_NCHUNK = _BPW // _CH
_NBUF = 3
_L = 16


def _logsm_body(mt_ref, out_ref):
    x = mt_ref[...]
    mx = jnp.max(x, axis=0, keepdims=True)
    s = x - mx
    lse = jnp.log(jnp.sum(jnp.exp(s), axis=0, keepdims=True))
    out_ref[...] = jnp.tile(s - lse, (_NW, 1))


def _make_table(matrix):
    return pl.pallas_call(
        _logsm_body,
        out_shape=jax.ShapeDtypeStruct((_NW * _M, _N), jnp.float32),
    )(matrix.T)


def _gather_body(table_hbm, xt_hbm, out_hbm, idx_v, rows_v, *sems):
    si = sems[0:_NBUF]
    sg = sems[_NBUF:2 * _NBUF]
    sw = sems[2 * _NBUF:3 * _NBUF]
    wid = lax.axis_index("s") * 2 + lax.axis_index("c")
    base = wid * _BPW
    bias = wid * _M

    def start_idx(b, i):
        return pltpu.async_copy(
            xt_hbm.at[pl.ds(base + i * _CH, _CH)], idx_v.at[b], si[b])

    def start_gather(b):
        return pltpu.async_copy(table_hbm.at[idx_v.at[b]], rows_v.at[b], sg[b])

    def start_write(b, i):
        return pltpu.async_copy(
            rows_v.at[b], out_hbm.at[pl.ds(base + i * _CH, _CH)], sw[b])

    def add_bias(b):
        def body(g, carry):
            o = g * _L
            idx_v[b, pl.ds(o, _L)] = idx_v[b, pl.ds(o, _L)] + bias
            return carry
        lax.fori_loop(0, _CH // _L, body, 0)

    h_idx = [None] * _NBUF
    h_g = [None] * _NBUF
    h_w = [None] * _NBUF
    for i in range(_NBUF):
        h_idx[i] = start_idx(i, i)
    for i in range(_NCHUNK):
        b = i % _NBUF
        h_idx[b].wait()
        add_bias(b)
        if i >= _NBUF:
            h_w[b].wait()
        h_g[b] = start_gather(b)
        if i >= 1:
            pb = (i - 1) % _NBUF
            h_g[pb].wait()
            h_w[pb] = start_write(pb, i - 1)
            if (i - 1) + _NBUF < _NCHUNK:
                h_idx[pb] = start_idx(pb, i - 1 + _NBUF)
    lb = (_NCHUNK - 1) % _NBUF
    h_g[lb].wait()
    h_w[lb] = start_write(lb, _NCHUNK - 1)
    for b in range(_NBUF):
        h_w[b].wait()


def kernel(matrix, x_t):
    table = _make_table(matrix)
    f = pl.kernel(
        _gather_body,
        out_type=jax.ShapeDtypeStruct((_B, _N), jnp.float32),
        mesh=plsc.VectorSubcoreMesh(core_axis_name="c", subcore_axis_name="s"),
        scratch_types=[
            pltpu.VMEM((_NBUF, _CH), jnp.int32),
            pltpu.VMEM((_NBUF, _CH, _N), jnp.float32),
        ] + [pltpu.SemaphoreType.DMA] * (3 * _NBUF),
        compiler_params=pltpu.CompilerParams(use_tc_tiling_on_sc=False),
    )
    return f(table, x_t)
